# baseline (speedup 1.0000x reference)
import functools

import jax
import jax.numpy as jnp
from jax import lax
from jax.experimental import pallas as pl
from jax.experimental.pallas import tpu as pltpu

N = 100000
E = 1600000
G = 64

_BR_N = 2000
_BR_E = 8000


def _node_proj_a(pos_ref, wu_ref, wv_ref, b_ref, u_ref, v_ref):
  p = pos_ref[...]
  u_ref[...] = jnp.dot(p, wu_ref[...], preferred_element_type=jnp.float32, precision=lax.Precision.HIGHEST) + b_ref[...]
  v_ref[...] = jnp.dot(p, wv_ref[...], preferred_element_type=jnp.float32, precision=lax.Precision.HIGHEST)


def _node_proj_b(h_ref, pos_ref, wh_ref, wp_ref, b_ref, u_ref, v_ref):
  t = jnp.dot(pos_ref[...], wp_ref[...], preferred_element_type=jnp.float32, precision=lax.Precision.HIGHEST)
  h = jnp.maximum(h_ref[...], 0.0)
  u_ref[...] = jnp.dot(h, wh_ref[...], preferred_element_type=jnp.float32, precision=lax.Precision.HIGHEST) + t + b_ref[...]
  v_ref[...] = t


def _edge_mlp(z_ref, w2_ref, b2_ref, m_ref):
  r = jnp.maximum(z_ref[...], 0.0)
  m = jnp.dot(r, w2_ref[...], preferred_element_type=jnp.float32, precision=lax.Precision.HIGHEST) + b2_ref[...]
  m_ref[...] = jnp.maximum(m, 0.0)


def _final(g_ref, wc_ref, bc_ref, o_ref):
  g = jnp.maximum(g_ref[...], 0.0)
  o_ref[...] = jnp.dot(g, wc_ref[...], preferred_element_type=jnp.float32, precision=lax.Precision.HIGHEST) + bc_ref[...]


def _run_node_proj_a(posP, Wu, Wv, b1):
  grid = (N // _BR_N,)
  return pl.pallas_call(
      _node_proj_a,
      grid=grid,
      in_specs=[
          pl.BlockSpec((_BR_N, 8), lambda i: (i, 0)),
          pl.BlockSpec((8, 32), lambda i: (0, 0)),
          pl.BlockSpec((8, 32), lambda i: (0, 0)),
          pl.BlockSpec((1, 32), lambda i: (0, 0)),
      ],
      out_specs=[
          pl.BlockSpec((_BR_N, 32), lambda i: (i, 0)),
          pl.BlockSpec((_BR_N, 32), lambda i: (i, 0)),
      ],
      out_shape=[
          jax.ShapeDtypeStruct((N, 32), jnp.float32),
          jax.ShapeDtypeStruct((N, 32), jnp.float32),
      ],
  )(posP, Wu, Wv, b1)


def _run_node_proj_b(h, posP, Wh, Wp, b1):
  grid = (N // _BR_N,)
  return pl.pallas_call(
      _node_proj_b,
      grid=grid,
      in_specs=[
          pl.BlockSpec((_BR_N, 32), lambda i: (i, 0)),
          pl.BlockSpec((_BR_N, 8), lambda i: (i, 0)),
          pl.BlockSpec((32, 32), lambda i: (0, 0)),
          pl.BlockSpec((8, 32), lambda i: (0, 0)),
          pl.BlockSpec((1, 32), lambda i: (0, 0)),
      ],
      out_specs=[
          pl.BlockSpec((_BR_N, 32), lambda i: (i, 0)),
          pl.BlockSpec((_BR_N, 32), lambda i: (i, 0)),
      ],
      out_shape=[
          jax.ShapeDtypeStruct((N, 32), jnp.float32),
          jax.ShapeDtypeStruct((N, 32), jnp.float32),
      ],
  )(h, posP, Wh, Wp, b1)


def _run_edge_mlp(z, W2, b2):
  grid = (E // _BR_E,)
  return pl.pallas_call(
      _edge_mlp,
      grid=grid,
      in_specs=[
          pl.BlockSpec((_BR_E, 32), lambda i: (i, 0)),
          pl.BlockSpec((32, 32), lambda i: (0, 0)),
          pl.BlockSpec((1, 32), lambda i: (0, 0)),
      ],
      out_specs=pl.BlockSpec((_BR_E, 32), lambda i: (i, 0)),
      out_shape=jax.ShapeDtypeStruct((E, 32), jnp.float32),
  )(z, W2, b2)


def _run_final(g, Wc, bc):
  return pl.pallas_call(
      _final,
      in_specs=[
          pl.BlockSpec((G, 32), lambda: (0, 0)),
          pl.BlockSpec((32, 2), lambda: (0, 0)),
          pl.BlockSpec((1, 2), lambda: (0, 0)),
      ],
      out_specs=pl.BlockSpec((G, 2), lambda: (0, 0)),
      out_shape=jax.ShapeDtypeStruct((G, 2), jnp.float32),
  )(g, Wc, bc)


def kernel(pos, edge_index, batch, W1a, b1a, W2a, b2a, W1b, b1b, W2b, b2b, Wc, bc):
  src, dst = edge_index[0], edge_index[1]
  posP = jnp.pad(pos, ((0, 0), (0, 5)))

  zeros5 = jnp.zeros((5, 32), jnp.float32)
  Wu_a = jnp.concatenate([W1a[0:3] + W1a[3:6], zeros5], axis=0)
  Wv_a = jnp.concatenate([W1a[3:6], zeros5], axis=0)
  Wh_b = W1b[0:32]
  Wp_b = jnp.concatenate([W1b[32:35], zeros5], axis=0)

  Ua, Va = _run_node_proj_a(posP, Wu_a, Wv_a, b1a.reshape(1, 32))
  z = Ua[src] - Va[dst]
  m = _run_edge_mlp(z, W2a, b2a.reshape(1, 32))
  h = jax.ops.segment_max(m, dst, num_segments=N)

  Ub, Vb = _run_node_proj_b(h, posP, Wh_b, Wp_b, b1b.reshape(1, 32))
  z = Ub[src] - Vb[dst]
  m = _run_edge_mlp(z, W2b, b2b.reshape(1, 32))
  h = jax.ops.segment_max(m, dst, num_segments=N)

  g = jax.ops.segment_max(jnp.maximum(h, 0.0), batch, num_segments=G)
  return _run_final(g, Wc, bc.reshape(1, 2))
